# baseline (speedup 1.0000x reference)
import functools

import jax
import jax.numpy as jnp
from jax import lax
from jax.experimental import pallas as pl
from jax.experimental.pallas import tpu as pltpu

E = 8
SEQ = 2048
D = 768
DFF = 3072
K = 4096

BT = 256
BF = 512
NF = DFF // BF
NBLK = K // BT + E
NP = NBLK * BT


def _gelu_tanh(v):
    return 0.5 * v * (1.0 + jnp.tanh(jnp.sqrt(2.0 / jnp.pi) * (v + 0.044715 * v ** 3)))


def _scores_body(x_ref, gw_ref, bias_ref, out_ref):
    lg = lax.dot_general(gw_ref[...], x_ref[...], (((1,), (1,)), ((), ())),
                         preferred_element_type=jnp.float32)
    out_ref[...] = jax.nn.sigmoid(lg + bias_ref[...])


def _scores(x_flat, gate_weight, expert_bias):
    return pl.pallas_call(
        _scores_body,
        out_shape=jax.ShapeDtypeStruct((E, SEQ), jnp.float32),
    )(x_flat, gate_weight, expert_bias)


def _mlp_body(be_ref, xg_ref, w1_ref, b1_ref, w2_ref, b2_ref, wp_ref, y_ref):
    f = pl.program_id(1)
    x_b = xg_ref[...]
    h = lax.dot_general(x_b, w1_ref[0], (((1,), (1,)), ((), ())),
                        preferred_element_type=jnp.float32)
    h = _gelu_tanh(h + b1_ref[0])
    part = lax.dot_general(h, w2_ref[0], (((1,), (1,)), ((), ())),
                           preferred_element_type=jnp.float32)

    @pl.when(f == 0)
    def _():
        y_ref[...] = jnp.zeros_like(y_ref)

    y_ref[...] += part

    @pl.when(f == NF - 1)
    def _():
        w = wp_ref[0, 0]
        y_ref[...] = (y_ref[...] + b2_ref[0]) * w[:, None]


def _grouped_mlp(xg, W1, b1, W2, b2, w_pad, blk_exp):
    grid_spec = pltpu.PrefetchScalarGridSpec(
        num_scalar_prefetch=1,
        grid=(NBLK, NF),
        in_specs=[
            pl.BlockSpec((BT, D), lambda m, f, be: (m, 0)),
            pl.BlockSpec((1, BF, D), lambda m, f, be: (be[m], f, 0)),
            pl.BlockSpec((1, 1, BF), lambda m, f, be: (be[m], 0, f)),
            pl.BlockSpec((1, D, BF), lambda m, f, be: (be[m], 0, f)),
            pl.BlockSpec((1, 1, D), lambda m, f, be: (be[m], 0, 0)),
            pl.BlockSpec((1, 1, BT), lambda m, f, be: (m, 0, 0)),
        ],
        out_specs=pl.BlockSpec((BT, D), lambda m, f, be: (m, 0)),
    )
    return pl.pallas_call(
        _mlp_body,
        grid_spec=grid_spec,
        out_shape=jax.ShapeDtypeStruct((NP, D), jnp.float32),
    )(blk_exp, xg, W1, b1.reshape(E, 1, DFF), W2, b2.reshape(E, 1, D),
      w_pad.reshape(NBLK, 1, BT))


def kernel(x, gate_weight, expert_bias, W1, b1, W2, b2):
    Bsz, seq, Dm = x.shape
    x_flat = x.reshape(-1, Dm)

    scores = _scores(x_flat, gate_weight, expert_bias)

    flat = scores.reshape(-1)
    vals, idx = lax.top_k(flat, K)
    e_sel = (idx // SEQ).astype(jnp.int32)
    t_sel = (idx % SEQ).astype(jnp.int32)

    counts = jnp.zeros((E,), jnp.int32).at[e_sel].add(1)
    cum_in = jnp.cumsum(counts)
    cum_ex = cum_in - counts

    order = jnp.argsort(e_sel)
    e_s = e_sel[order]
    t_s = t_sel[order]
    w_s = vals[order]

    nblk_e = (counts + BT - 1) // BT
    cnb_in = jnp.cumsum(nblk_e)
    cnb_ex = cnb_in - nblk_e
    blk_start = BT * cnb_ex

    pos = blk_start[e_s] + (jnp.arange(K, dtype=jnp.int32) - cum_ex[e_s])
    tok_pad = jnp.zeros((NP,), jnp.int32).at[pos].set(t_s)
    w_pad = jnp.zeros((NP,), jnp.float32).at[pos].set(w_s)

    used = cnb_in[-1]
    bids = jnp.arange(NBLK, dtype=jnp.int32)
    blk_exp = jnp.searchsorted(cnb_in, bids, side='right').astype(jnp.int32)
    blk_exp = jnp.where(bids < used, blk_exp, 0)

    xg = x_flat[tok_pad]
    y = _grouped_mlp(xg, W1, b1, W2, b2, w_pad, blk_exp)

    out = jnp.zeros((SEQ, Dm), jnp.float32).at[tok_pad].add(y)

    token_each_expert = counts.astype(jnp.float32) / float(K)
    ones_like_mean = jnp.ones((E,), jnp.float32)
    return (out.reshape(Bsz, seq, Dm), token_each_expert, ones_like_mean)


# speedup vs baseline: 1.2682x; 1.2682x over previous
import functools

import jax
import jax.numpy as jnp
from jax import lax
from jax.experimental import pallas as pl
from jax.experimental.pallas import tpu as pltpu

E = 8
SEQ = 2048
D = 768
DFF = 3072
K = 4096

BT = 256
BF = 512
NF = DFF // BF
NBLK = K // BT + E
NP = NBLK * BT


def _gelu_tanh(v):
    return 0.5 * v * (1.0 + jnp.tanh(jnp.sqrt(2.0 / jnp.pi) * (v + 0.044715 * v ** 3)))


def _scores_body(x_ref, gw_ref, bias_ref, out_ref):
    lg = lax.dot_general(gw_ref[...], x_ref[...], (((1,), (1,)), ((), ())),
                         preferred_element_type=jnp.float32)
    out_ref[...] = jax.nn.sigmoid(lg + bias_ref[...])


def _scores(x_flat, gate_weight, expert_bias):
    return pl.pallas_call(
        _scores_body,
        out_shape=jax.ShapeDtypeStruct((E, SEQ), jnp.float32),
    )(x_flat, gate_weight, expert_bias)


def _mlp_body(be_ref, xg_ref, w1_ref, b1_ref, w2_ref, b2_ref, wp_ref, y_ref):
    x_b = xg_ref[...]
    h = lax.dot_general(x_b, w1_ref[0], (((1,), (1,)), ((), ())),
                        preferred_element_type=jnp.float32)
    h = _gelu_tanh(h + b1_ref[0])
    part = lax.dot_general(h, w2_ref[0], (((1,), (1,)), ((), ())),
                           preferred_element_type=jnp.float32)
    w = wp_ref[0, 0]
    y_ref[...] = (part + b2_ref[0]) * w[:, None]


def _grouped_mlp(xg, W1, b1, W2, b2, w_pad, blk_exp):
    grid_spec = pltpu.PrefetchScalarGridSpec(
        num_scalar_prefetch=1,
        grid=(NBLK,),
        in_specs=[
            pl.BlockSpec((BT, D), lambda m, be: (m, 0)),
            pl.BlockSpec((1, DFF, D), lambda m, be: (be[m], 0, 0)),
            pl.BlockSpec((1, 1, DFF), lambda m, be: (be[m], 0, 0)),
            pl.BlockSpec((1, D, DFF), lambda m, be: (be[m], 0, 0)),
            pl.BlockSpec((1, 1, D), lambda m, be: (be[m], 0, 0)),
            pl.BlockSpec((1, 1, BT), lambda m, be: (m, 0, 0)),
        ],
        out_specs=pl.BlockSpec((BT, D), lambda m, be: (m, 0)),
    )
    return pl.pallas_call(
        _mlp_body,
        grid_spec=grid_spec,
        out_shape=jax.ShapeDtypeStruct((NP, D), jnp.float32),
    )(blk_exp, xg, W1, b1.reshape(E, 1, DFF), W2, b2.reshape(E, 1, D),
      w_pad.reshape(NBLK, 1, BT))


def kernel(x, gate_weight, expert_bias, W1, b1, W2, b2):
    Bsz, seq, Dm = x.shape
    x_flat = x.reshape(-1, Dm)

    scores = _scores(x_flat, gate_weight, expert_bias)

    flat = scores.reshape(-1)
    vals, idx = lax.top_k(flat, K)
    e_sel = (idx // SEQ).astype(jnp.int32)
    t_sel = (idx % SEQ).astype(jnp.int32)

    counts = jnp.zeros((E,), jnp.int32).at[e_sel].add(1)
    cum_in = jnp.cumsum(counts)
    cum_ex = cum_in - counts

    order = jnp.argsort(e_sel)
    e_s = e_sel[order]
    t_s = t_sel[order]
    w_s = vals[order]

    nblk_e = (counts + BT - 1) // BT
    cnb_in = jnp.cumsum(nblk_e)
    cnb_ex = cnb_in - nblk_e
    blk_start = BT * cnb_ex

    pos = blk_start[e_s] + (jnp.arange(K, dtype=jnp.int32) - cum_ex[e_s])
    tok_pad = jnp.zeros((NP,), jnp.int32).at[pos].set(t_s)
    w_pad = jnp.zeros((NP,), jnp.float32).at[pos].set(w_s)

    used = cnb_in[-1]
    bids = jnp.arange(NBLK, dtype=jnp.int32)
    blk_exp = jnp.searchsorted(cnb_in, bids, side='right').astype(jnp.int32)
    blk_exp = jnp.where(bids < used, blk_exp, 0)

    xg = x_flat[tok_pad]
    y = _grouped_mlp(xg, W1, b1, W2, b2, w_pad, blk_exp)

    out = jnp.zeros((SEQ, Dm), jnp.float32).at[tok_pad].add(y)

    token_each_expert = counts.astype(jnp.float32) / float(K)
    ones_like_mean = jnp.ones((E,), jnp.float32)
    return (out.reshape(Bsz, seq, Dm), token_each_expert, ones_like_mean)
